# baseline (speedup 1.0000x reference)
import functools

import jax
import jax.numpy as jnp
from jax import lax
from jax.experimental import pallas as pl
from jax.experimental.pallas import tpu as pltpu
from jax.experimental.pallas import tpu_sc as plsc

NC = 2
NS = 16
NW = NC * NS
BATCH = 16384
LATENT = 64
CHUNK = BATCH // NW
NIDX = 4
IDXW = CHUNK // NIDX
NBLK = CHUNK // 16

_mesh = plsc.VectorSubcoreMesh(core_axis_name="c", subcore_axis_name="s")

_scratch_types = [
    pltpu.VMEM((NIDX, IDXW), jnp.int32),
    pltpu.VMEM((NIDX, IDXW), jnp.int32),
    pltpu.VMEM((LATENT, CHUNK), jnp.float32),
    pltpu.VMEM((LATENT, CHUNK), jnp.float32),
    pltpu.VMEM((NIDX, IDXW), jnp.float32),
    pltpu.VMEM((NIDX, IDXW), jnp.float32),
    pltpu.VMEM((CHUNK,), jnp.float32),
    pltpu.SemaphoreType.DMA,
]


def _mf_body(users_hbm, items_hbm, pt_hbm, qt_hbm, bu_hbm, bi_hbm, out_hbm,
             uidx, iidx, ptv, qtv, bu, bi, outb, sem):
    wid = lax.axis_index("s") * NC + lax.axis_index("c")

    pltpu.sync_copy(users_hbm.at[wid], uidx)
    pltpu.sync_copy(items_hbm.at[wid], iidx)

    bias_copies = []
    for j in range(NIDX):
        bias_copies.append(pltpu.async_copy(bu_hbm.at[uidx.at[j]], bu.at[j], sem))
        bias_copies.append(pltpu.async_copy(bi_hbm.at[iidx.at[j]], bi.at[j], sem))

    def fire(d, carry):
        for j in range(NIDX):
            pltpu.async_copy(pt_hbm.at[d].at[uidx.at[j]],
                             ptv.at[d, pl.ds(j * IDXW, IDXW)], sem)
            pltpu.async_copy(qt_hbm.at[d].at[iidx.at[j]],
                             qtv.at[d, pl.ds(j * IDXW, IDXW)], sem)
        return carry

    lax.fori_loop(0, LATENT, fire, 0)

    for c in bias_copies:
        c.wait()
    pltpu.make_async_copy(pt_hbm.at[:, pl.ds(0, CHUNK)], ptv, sem).wait()
    pltpu.make_async_copy(qt_hbm.at[:, pl.ds(0, CHUNK)], qtv, sem).wait()

    def blk_body(blk, carry):
        off = blk * 16
        acc = ptv[0, pl.ds(off, 16)] * qtv[0, pl.ds(off, 16)]
        for d in range(1, LATENT):
            acc = acc + ptv[d, pl.ds(off, 16)] * qtv[d, pl.ds(off, 16)]
        j = blk // (IDXW // 16)
        boff = (blk % (IDXW // 16)) * 16
        acc = acc + bu[j, pl.ds(boff, 16)] + bi[j, pl.ds(boff, 16)]
        outb[pl.ds(off, 16)] = acc
        return carry

    lax.fori_loop(0, NBLK, blk_body, 0)

    pltpu.sync_copy(outb, out_hbm.at[pl.ds(wid * CHUNK, CHUNK)])


_mf_kernel = functools.partial(
    pl.kernel,
    out_type=jax.ShapeDtypeStruct((BATCH,), jnp.float32),
    mesh=_mesh,
    scratch_types=_scratch_types,
    compiler_params=pltpu.CompilerParams(use_tc_tiling_on_sc=False),
)(_mf_body)


def kernel(users, items, P, Q, user_bias, item_bias):
    users_r = users.reshape(NW, NIDX, IDXW)
    items_r = items.reshape(NW, NIDX, IDXW)
    pt = P.T
    qt = Q.T
    bu_flat = user_bias.reshape(-1)
    bi_flat = item_bias.reshape(-1)
    return _mf_kernel(users_r, items_r, pt, qt, bu_flat, bi_flat)


# speedup vs baseline: 8.5820x; 8.5820x over previous
import functools

import jax
import jax.numpy as jnp
from jax import lax
from jax.experimental import pallas as pl
from jax.experimental.pallas import tpu as pltpu
from jax.experimental.pallas import tpu_sc as plsc

NC = 2
NS = 16
NW = NC * NS
BATCH = 16384
LATENT = 64
ROWW = 128
NROW = 500000
CHUNK = BATCH // NW
NIDX = 4
IDXW = CHUNK // NIDX
HALF = NIDX // 2

_mesh = plsc.VectorSubcoreMesh(core_axis_name="c", subcore_axis_name="s")

_dot_scratch = [
    pltpu.VMEM((NIDX, IDXW), jnp.int32),
    pltpu.VMEM((NIDX, IDXW), jnp.int32),
    pltpu.VMEM((NIDX, IDXW), jnp.int32),
    pltpu.VMEM((NIDX, IDXW), jnp.int32),
    pltpu.VMEM((HALF, IDXW, ROWW), jnp.float32),
    pltpu.VMEM((HALF, IDXW, ROWW), jnp.float32),
    pltpu.VMEM((CHUNK,), jnp.float32),
    pltpu.SemaphoreType.DMA,
]


def _dot_body(uid_hbm, iid_hbm, urow_hbm, irow_hbm, p_hbm, q_hbm, out_hbm,
              uid, iid, urow, irow, pm, qm, outb, sem):
    wid = lax.axis_index("s") * NC + lax.axis_index("c")

    pltpu.sync_copy(uid_hbm.at[wid], uid)
    pltpu.sync_copy(iid_hbm.at[wid], iid)
    pltpu.sync_copy(urow_hbm.at[wid], urow)
    pltpu.sync_copy(irow_hbm.at[wid], irow)

    one = jnp.int32(1)
    lane = lax.iota(jnp.int32, 16)
    perms = [lane ^ s for s in (1, 2, 4, 8)]
    picks = [(lane & s) == 0 for s in (1, 2, 4, 8)]

    def _take(v, idx):
        return jnp.take_along_axis(v, idx, axis=0)

    for h in range(2):
        copies = []
        for j in range(HALF):
            copies.append(pltpu.async_copy(
                p_hbm.at[urow.at[h * HALF + j]], pm.at[j], sem))
            copies.append(pltpu.async_copy(
                q_hbm.at[irow.at[h * HALF + j]], qm.at[j], sem))
        for c in copies:
            c.wait()

        def blk_body(b, carry):
            j = b // (IDXW // 16)
            off = (b % (IDXW // 16)) * 16
            uv = uid[h * HALF + j, pl.ds(off, 16)]
            iv = iid[h * HALF + j, pl.ds(off, 16)]
            vs = []
            for r in range(16):
                row = off + r
                uh = (uv[r] & one) * 64
                ih = (iv[r] & one) * 64
                acc = pm[j, row, pl.ds(uh, 16)] * qm[j, row, pl.ds(ih, 16)]
                for k in range(1, LATENT // 16):
                    acc = acc + (pm[j, row, pl.ds(uh + 16 * k, 16)]
                                 * qm[j, row, pl.ds(ih + 16 * k, 16)])
                vs.append(acc)
            for lvl in range(4):
                nxt = []
                for i in range(0, len(vs), 2):
                    a2, b2 = vs[i], vs[i + 1]
                    ap = a2 + _take(a2, perms[lvl])
                    bp = b2 + _take(b2, perms[lvl])
                    nxt.append(jnp.where(picks[lvl], ap, bp))
                vs = nxt
            outb[pl.ds(h * (CHUNK // 2) + b * 16, 16)] = vs[0]
            return carry

        lax.fori_loop(0, HALF * IDXW // 16, blk_body, 0)

    pltpu.sync_copy(outb, out_hbm.at[pl.ds(wid * CHUNK, CHUNK)])


_bias_scratch = [
    pltpu.VMEM((NIDX, IDXW), jnp.int32),
    pltpu.VMEM((NIDX, IDXW), jnp.int32),
    pltpu.VMEM((NIDX, IDXW), jnp.float32),
    pltpu.VMEM((NIDX, IDXW), jnp.float32),
    pltpu.VMEM((CHUNK,), jnp.float32),
    pltpu.SemaphoreType.DMA,
]


def _bias_body(uid_hbm, iid_hbm, part_hbm, bu_hbm, bi_hbm, out_hbm,
               uid, iid, bu, bi, outb, sem):
    wid = lax.axis_index("s") * NC + lax.axis_index("c")
    base = wid * CHUNK

    pltpu.sync_copy(uid_hbm.at[wid], uid)
    pltpu.sync_copy(iid_hbm.at[wid], iid)
    pltpu.sync_copy(part_hbm.at[pl.ds(base, CHUNK)], outb)

    copies = []
    for j in range(NIDX):
        copies.append(pltpu.async_copy(bu_hbm.at[uid.at[j]], bu.at[j], sem))
        copies.append(pltpu.async_copy(bi_hbm.at[iid.at[j]], bi.at[j], sem))
    for c in copies:
        c.wait()

    def blk_body(blk, carry):
        j = blk // (IDXW // 16)
        off = (blk % (IDXW // 16)) * 16
        o = blk * 16
        outb[pl.ds(o, 16)] = (outb[pl.ds(o, 16)]
                              + bu[j, pl.ds(off, 16)] + bi[j, pl.ds(off, 16)])
        return carry

    lax.fori_loop(0, CHUNK // 16, blk_body, 0)

    pltpu.sync_copy(outb, out_hbm.at[pl.ds(base, CHUNK)])


_dot_kernel = functools.partial(
    pl.kernel,
    out_type=jax.ShapeDtypeStruct((BATCH,), jnp.float32),
    mesh=_mesh,
    scratch_types=_dot_scratch,
    compiler_params=pltpu.CompilerParams(use_tc_tiling_on_sc=True),
)(_dot_body)

_bias_kernel = functools.partial(
    pl.kernel,
    out_type=jax.ShapeDtypeStruct((BATCH,), jnp.float32),
    mesh=_mesh,
    scratch_types=_bias_scratch,
    compiler_params=pltpu.CompilerParams(use_tc_tiling_on_sc=False),
)(_bias_body)


def kernel(users, items, P, Q, user_bias, item_bias):
    uid = users.reshape(NW, NIDX, IDXW)
    iid = items.reshape(NW, NIDX, IDXW)
    urow = (users // 2).reshape(NW, NIDX, IDXW)
    irow = (items // 2).reshape(NW, NIDX, IDXW)
    p2 = P.reshape(NROW, ROWW)
    q2 = Q.reshape(NROW, ROWW)
    part = _dot_kernel(uid, iid, urow, irow, p2, q2)
    bu_flat = user_bias.reshape(-1)
    bi_flat = item_bias.reshape(-1)
    return _bias_kernel(uid, iid, part, bu_flat, bi_flat)


# speedup vs baseline: 9.1086x; 1.0614x over previous
import functools

import jax
import jax.numpy as jnp
from jax import lax
from jax.experimental import pallas as pl
from jax.experimental.pallas import tpu as pltpu
from jax.experimental.pallas import tpu_sc as plsc

NC = 2
NS = 16
NW = NC * NS
BATCH = 16384
LATENT = 64
ROWW = 128
NROW = 1000000
CHUNK = BATCH // NW
NIDX = 4
IDXW = CHUNK // NIDX
HALF = NIDX // 2

_mesh = plsc.VectorSubcoreMesh(core_axis_name="c", subcore_axis_name="s")

_dot_scratch = [
    pltpu.VMEM((NIDX, IDXW), jnp.int32),
    pltpu.VMEM((NIDX, IDXW), jnp.int32),
    pltpu.VMEM((HALF, IDXW, ROWW), jnp.float32),
    pltpu.VMEM((HALF, IDXW, ROWW), jnp.float32),
    pltpu.VMEM((CHUNK,), jnp.float32),
    pltpu.SemaphoreType.DMA,
]


def _dot_body(uid_hbm, iid_hbm, p_hbm, q_hbm, out_hbm,
              uid, iid, pm, qm, outb, sem):
    wid = lax.axis_index("s") * NC + lax.axis_index("c")

    pltpu.sync_copy(uid_hbm.at[wid], uid)
    pltpu.sync_copy(iid_hbm.at[wid], iid)

    lane = lax.iota(jnp.int32, 16)
    perms = [lane ^ s for s in (1, 2, 4, 8)]
    picks = [(lane & s) == 0 for s in (1, 2, 4, 8)]

    def _take(v, idx):
        return jnp.take_along_axis(v, idx, axis=0)

    for h in range(2):
        copies = []
        for j in range(HALF):
            copies.append(pltpu.async_copy(
                p_hbm.at[uid.at[h * HALF + j]], pm.at[j], sem))
            copies.append(pltpu.async_copy(
                q_hbm.at[iid.at[h * HALF + j]], qm.at[j], sem))
        for c in copies:
            c.wait()

        def blk_body(b, carry):
            j = b // (IDXW // 16)
            off = (b % (IDXW // 16)) * 16
            vs = []
            for r in range(16):
                row = off + r
                acc = pm[j, row, pl.ds(0, 16)] * qm[j, row, pl.ds(0, 16)]
                for k in range(1, LATENT // 16):
                    acc = acc + (pm[j, row, pl.ds(16 * k, 16)]
                                 * qm[j, row, pl.ds(16 * k, 16)])
                vs.append(acc)
            for lvl in range(4):
                nxt = []
                for i in range(0, len(vs), 2):
                    a2, b2 = vs[i], vs[i + 1]
                    ap = a2 + _take(a2, perms[lvl])
                    bp = b2 + _take(b2, perms[lvl])
                    nxt.append(jnp.where(picks[lvl], ap, bp))
                vs = nxt
            outb[pl.ds(h * (CHUNK // 2) + b * 16, 16)] = vs[0]
            return carry

        lax.fori_loop(0, HALF * IDXW // 16, blk_body, 0)

    pltpu.sync_copy(outb, out_hbm.at[pl.ds(wid * CHUNK, CHUNK)])


_bias_scratch = [
    pltpu.VMEM((NIDX, IDXW), jnp.int32),
    pltpu.VMEM((NIDX, IDXW), jnp.int32),
    pltpu.VMEM((NIDX, IDXW), jnp.float32),
    pltpu.VMEM((NIDX, IDXW), jnp.float32),
    pltpu.VMEM((CHUNK,), jnp.float32),
    pltpu.SemaphoreType.DMA,
]


def _bias_body(uid_hbm, iid_hbm, part_hbm, bu_hbm, bi_hbm, out_hbm,
               uid, iid, bu, bi, outb, sem):
    wid = lax.axis_index("s") * NC + lax.axis_index("c")
    base = wid * CHUNK

    pltpu.sync_copy(uid_hbm.at[wid], uid)
    pltpu.sync_copy(iid_hbm.at[wid], iid)
    pltpu.sync_copy(part_hbm.at[pl.ds(base, CHUNK)], outb)

    copies = []
    for j in range(NIDX):
        copies.append(pltpu.async_copy(bu_hbm.at[uid.at[j]], bu.at[j], sem))
        copies.append(pltpu.async_copy(bi_hbm.at[iid.at[j]], bi.at[j], sem))
    for c in copies:
        c.wait()

    def blk_body(blk, carry):
        j = blk // (IDXW // 16)
        off = (blk % (IDXW // 16)) * 16
        o = blk * 16
        outb[pl.ds(o, 16)] = (outb[pl.ds(o, 16)]
                              + bu[j, pl.ds(off, 16)] + bi[j, pl.ds(off, 16)])
        return carry

    lax.fori_loop(0, CHUNK // 16, blk_body, 0)

    pltpu.sync_copy(outb, out_hbm.at[pl.ds(base, CHUNK)])


_dot_kernel = functools.partial(
    pl.kernel,
    out_type=jax.ShapeDtypeStruct((BATCH,), jnp.float32),
    mesh=_mesh,
    scratch_types=_dot_scratch,
    compiler_params=pltpu.CompilerParams(use_tc_tiling_on_sc=True),
)(_dot_body)

_bias_kernel = functools.partial(
    pl.kernel,
    out_type=jax.ShapeDtypeStruct((BATCH,), jnp.float32),
    mesh=_mesh,
    scratch_types=_bias_scratch,
    compiler_params=pltpu.CompilerParams(use_tc_tiling_on_sc=False),
)(_bias_body)


def kernel(users, items, P, Q, user_bias, item_bias):
    uid = users.reshape(NW, NIDX, IDXW)
    iid = items.reshape(NW, NIDX, IDXW)
    p2 = jnp.pad(P, ((0, 0), (0, ROWW - LATENT)))
    q2 = jnp.pad(Q, ((0, 0), (0, ROWW - LATENT)))
    part = _dot_kernel(uid, iid, p2, q2)
    bu_flat = user_bias.reshape(-1)
    bi_flat = item_bias.reshape(-1)
    return _bias_kernel(uid, iid, part, bu_flat, bi_flat)
